# baseline (speedup 1.0000x reference)
import numpy as np
import jax
import jax.numpy as jnp
from jax.experimental import pallas as pl
from jax.experimental.pallas import tpu as pltpu

_Hl, _Wl = 14, 14
_SCALE = 16
_R_MAX = 3
_ALPHA_DYN = 2.0
_Hh, _Wh = _Hl * _SCALE, _Wl * _SCALE
_NPIX = _Hh * _Wh
_EXT = _Wl + 2 * _R_MAX
_NR = 7 * _EXT
_NRP = 144
_P = _SCALE * _Wh
_NCELL = _Hl * _NRP
_LOG2E = float(np.log2(np.e))


def _resize_mat(dst, src):
    m = np.zeros((dst, src), np.float32)
    for y in range(dst):
        u = (y + 0.5) * src / dst - 0.5
        i0 = int(np.floor(u))
        f = u - i0
        m[y, min(max(i0, 0), src - 1)] += 1.0 - f
        m[y, min(max(i0 + 1, 0), src - 1)] += f
    return m


def _build_static():
    dys = np.arange(-_R_MAX, _R_MAX + 1)
    ext_j = np.arange(-_R_MAX, _Wl + _R_MAX)
    ts = np.arange(_Hl)
    iu = np.broadcast_to(ts[:, None, None] + dys[None, :, None],
                         (_Hl, 7, _EXT)).reshape(_Hl, _NR)
    ju = np.broadcast_to(ext_j[None, None, :],
                         (_Hl, 7, _EXT)).reshape(_Hl, _NR)
    npad = _NRP - _NR
    padi = np.full((_Hl, npad), 10 ** 4, np.int64)
    iu = np.concatenate([iu, padi], 1)
    ju = np.concatenate([ju, padi], 1)
    icl = np.clip(iu, 0, _Hl - 1)
    jcl = np.clip(ju, 0, _Wl - 1)
    pad_mask = np.zeros((_Hl, _NRP), np.float32)
    pad_mask[:, _NR:] = 1.0

    ohi = np.zeros((_NCELL, _Hl), np.float32)
    ohj = np.zeros((_NCELL, _Wl), np.float32)
    r = np.arange(_NCELL)
    ohi[r, icl.reshape(-1)] = 1.0
    ohj[r, jcl.reshape(-1)] = 1.0

    band = np.repeat(ts, _NRP)
    cxv = (jcl.reshape(-1) + 0.5) * _SCALE - 0.5 - 112.0
    cyl = (icl.reshape(-1) + 0.5) * _SCALE - 0.5 - _SCALE * band
    dy2 = (iu.reshape(-1) - band).astype(np.float64) ** 2
    sqc = dy2 + ju.reshape(-1).astype(np.float64) ** 2
    sqc = np.where(pad_mask.reshape(-1) > 0, 1e8, sqc)
    jm2 = -2.0 * ju.reshape(-1)
    jm2 = np.where(pad_mask.reshape(-1) > 0, 0.0, jm2)
    stat = np.stack([cxv, cyl, sqc, jm2,
                     np.zeros(_NCELL), np.zeros(_NCELL),
                     np.zeros(_NCELL), np.zeros(_NCELL)], 1).astype(np.float32)

    flat = (icl * _Wl + jcl).reshape(-1)
    ohf = np.zeros((_Hl, _Hl * _Wl, _NRP), np.float32)
    b = np.repeat(ts, _NRP)
    c = np.tile(np.arange(_NRP), _Hl)
    live = (pad_mask.reshape(-1) == 0)
    ohf[b[live], flat[live], c[live]] = 1.0

    bh14 = _resize_mat(_Hl, _Hh)
    bwt = _resize_mat(_Wl, _Wh).T
    ah = _resize_mat(_Hh, _Hl)
    return ohi, ohj, stat, ohf, bh14, bwt, ah


_OHI, _OHJ, _STAT, _OHF, _BH14, _BWT, _AH = _build_static()


def _jbu_tile(r2_ref, guide3_ref, g672_ref, sxr_ref, syr_ref, thr_ref,
              srr_ref, feat_ref, ohf_ref, ohi_ref, ohj_ref, stat_ref,
              bh_ref, bwt_ref, out_ref, rl_scr, rs_scr):
    g = pl.program_id(0)
    f32 = jnp.float32

    @pl.when(g == 0)
    def _build_tables():
        ohi = ohi_ref[...]
        ohj = ohj_ref[...]

        def gath(m14):
            t = jnp.dot(ohi, m14, preferred_element_type=f32,
                        precision=jax.lax.Precision.HIGHEST)
            return jnp.sum(t * ohj, axis=1, keepdims=True)

        gsx = gath(sxr_ref[...])
        gsy = gath(syr_ref[...])
        gth = gath(thr_ref[...])
        gsr = gath(srr_ref[...])
        sxm = jnp.maximum(jnp.exp(gsx), 1e-6)
        sym = jnp.maximum(jnp.exp(gsy), 1e-6)
        srm = jnp.maximum(jnp.exp(gsr), 1e-6)
        isx = _LOG2E / (2.0 * sxm * sxm + 1e-8)
        isy = _LOG2E / (2.0 * sym * sym + 1e-8)
        isr = _LOG2E / (2.0 * srm * srm + 1e-8)
        th = jnp.pi * jnp.tanh(gth)
        ct = jnp.cos(th)
        st = jnp.sin(th)
        qa = ct * ct * isx + st * st * isy
        qb = 2.0 * ct * st * (isx - isy)
        qc = st * st * isx + ct * ct * isy

        bh = bh_ref[...]
        bwt = bwt_ref[...]
        gls = []
        for ch in range(3):
            gc = g672_ref[ch * _Hh:(ch + 1) * _Hh, :]
            glr = jnp.dot(bh, jnp.dot(gc, bwt, preferred_element_type=f32,
                                      precision=jax.lax.Precision.HIGHEST),
                          preferred_element_type=f32,
                          precision=jax.lax.Precision.HIGHEST)
            gls.append(gath(glr))
        gl0, gl1, gl2 = gls

        stat = stat_ref[...]
        cxv = stat[:, 0:1]
        cyl = stat[:, 1:2]
        sqc = stat[:, 2:3]
        jm2 = stat[:, 3:4]
        z = jnp.zeros_like(cxv)
        one = z + 1.0

        col0 = -(qa * cxv * cxv + qb * cxv * cyl + qc * cyl * cyl
                 + isr * (gl0 * gl0 + gl1 * gl1 + gl2 * gl2))
        rl_scr[...] = jnp.concatenate([
            col0,
            2.0 * qa * cxv + qb * cyl,
            2.0 * qc * cyl + qb * cxv,
            z,
            -qa,
            -qb,
            -qc,
            2.0 * isr * gl0,
            2.0 * isr * gl1,
            2.0 * isr * gl2,
            -isr,
            z, z, z, z, z], axis=1)
        rs_scr[...] = jnp.concatenate([
            sqc, z, z, jm2, z, z, z, z, z, z, z,
            one,
            -one,
            z, z, z], axis=1)

    li = jax.lax.broadcasted_iota(jnp.int32, (1, _P), 1)
    xi = li % _Wh
    yl = li // _Wh
    xv = xi.astype(f32) - 112.0
    ylv = yl.astype(f32)
    vcf = (xi // _SCALE).astype(f32)
    gh = guide3_ref[...]
    gh0 = gh[0:1, :]
    gh1 = gh[1:2, :]
    gh2 = gh[2:3, :]
    r2 = r2_ref[...].reshape(1, _P)
    ones = jnp.zeros((1, _P), f32) + 1.0
    zr = jnp.zeros((1, _P), f32)
    cf = jnp.concatenate([
        ones, xv, ylv, vcf, xv * xv, xv * ylv, ylv * ylv,
        gh0, gh1, gh2, gh0 * gh0 + gh1 * gh1 + gh2 * gh2,
        vcf * vcf, r2, zr, zr, zr], axis=0)

    hi = jax.lax.Precision.HIGHEST
    rl = rl_scr[pl.ds(g * _NRP, _NRP), :]
    rs = rs_scr[pl.ds(g * _NRP, _NRP), :]
    lw2 = jnp.dot(rl, cf, preferred_element_type=f32, precision=hi)
    sqmr = jnp.dot(rs, cf, preferred_element_type=f32, precision=hi)
    lwm = lw2 - jnp.maximum(sqmr - 0.5, 0.0) * 1e30
    m = jnp.max(lwm, axis=0, keepdims=True)
    s = jnp.exp2(lwm - m)
    den = jnp.sum(s, axis=0, keepdims=True)
    featb = jnp.dot(feat_ref[...], ohf_ref[...].reshape(_Hl * _Wl, _NRP),
                    preferred_element_type=f32, precision=hi)
    num = jnp.dot(featb, s, preferred_element_type=f32, precision=hi)
    out_ref[...] = num / den


def kernel(feat_lr, guide_hr, sx_raw, sy_raw, th_raw, sr_raw):
    f32 = jnp.float32
    nc = feat_lr.shape[1]

    smax = jnp.exp(jnp.maximum(sx_raw, sy_raw))[0, 0]
    ah = jnp.asarray(_AH)
    hi = jax.lax.Precision.HIGHEST
    sig = jnp.dot(ah, jnp.dot(smax, ah.T, precision=hi),
                  precision=hi)
    rm = jnp.clip(jnp.ceil(_ALPHA_DYN * sig), 1, _R_MAX)
    r2 = (rm * rm).reshape(_Hl, 1, _P)

    guide3 = guide_hr[0].astype(f32).reshape(3, _NPIX)
    g672 = guide_hr[0].astype(f32).reshape(3 * _Hh, _Wh)
    feat196 = feat_lr[0].astype(f32).reshape(nc, _Hl * _Wl)

    out = pl.pallas_call(
        _jbu_tile,
        grid=(_Hl,),
        in_specs=[
            pl.BlockSpec((1, 1, _P), lambda g: (g, 0, 0)),
            pl.BlockSpec((3, _P), lambda g: (0, g)),
            pl.BlockSpec((3 * _Hh, _Wh), lambda g: (0, 0)),
            pl.BlockSpec((_Hl, _Wl), lambda g: (0, 0)),
            pl.BlockSpec((_Hl, _Wl), lambda g: (0, 0)),
            pl.BlockSpec((_Hl, _Wl), lambda g: (0, 0)),
            pl.BlockSpec((_Hl, _Wl), lambda g: (0, 0)),
            pl.BlockSpec((nc, _Hl * _Wl), lambda g: (0, 0)),
            pl.BlockSpec((1, _Hl * _Wl, _NRP), lambda g: (g, 0, 0)),
            pl.BlockSpec((_NCELL, _Hl), lambda g: (0, 0)),
            pl.BlockSpec((_NCELL, _Wl), lambda g: (0, 0)),
            pl.BlockSpec((_NCELL, 8), lambda g: (0, 0)),
            pl.BlockSpec((_Hl, _Hh), lambda g: (0, 0)),
            pl.BlockSpec((_Hh, _Wl), lambda g: (0, 0)),
        ],
        out_specs=pl.BlockSpec((nc, _P), lambda g: (0, g)),
        out_shape=jax.ShapeDtypeStruct((nc, _NPIX), f32),
        scratch_shapes=[
            pltpu.VMEM((_NCELL, 16), f32),
            pltpu.VMEM((_NCELL, 16), f32),
        ],
    )(r2, guide3, g672, sx_raw[0, 0], sy_raw[0, 0], th_raw[0, 0],
      sr_raw[0, 0], feat196, jnp.asarray(_OHF), jnp.asarray(_OHI),
      jnp.asarray(_OHJ), jnp.asarray(_STAT), jnp.asarray(_BH14),
      jnp.asarray(_BWT))

    return out.reshape(1, nc, _Hh, _Wh).astype(feat_lr.dtype)


# speedup vs baseline: 1.1003x; 1.1003x over previous
import numpy as np
import jax
import jax.numpy as jnp
from jax.experimental import pallas as pl
from jax.experimental.pallas import tpu as pltpu

_Hl, _Wl = 14, 14
_SCALE = 16
_R_MAX = 3
_ALPHA_DYN = 2.0
_Hh, _Wh = _Hl * _SCALE, _Wl * _SCALE
_NPIX = _Hh * _Wh
_EXT = _Wl + 2 * _R_MAX
_NR = 7 * _EXT
_NRP = 144
_P = _SCALE * _Wh
_NCELL = _Hl * _NRP
_NMAP = 8
_NFA = 104
_LOG2E = float(np.log2(np.e))


def _resize_mat(dst, src):
    m = np.zeros((dst, src), np.float32)
    for y in range(dst):
        u = (y + 0.5) * src / dst - 0.5
        i0 = int(np.floor(u))
        f = u - i0
        m[y, min(max(i0, 0), src - 1)] += 1.0 - f
        m[y, min(max(i0 + 1, 0), src - 1)] += f
    return m


def _build_static():
    dys = np.arange(-_R_MAX, _R_MAX + 1)
    ext_j = np.arange(-_R_MAX, _Wl + _R_MAX)
    ts = np.arange(_Hl)
    iu = np.broadcast_to(ts[:, None, None] + dys[None, :, None],
                         (_Hl, 7, _EXT)).reshape(_Hl, _NR)
    ju = np.broadcast_to(ext_j[None, None, :],
                         (_Hl, 7, _EXT)).reshape(_Hl, _NR)
    npad = _NRP - _NR
    padi = np.full((_Hl, npad), 10 ** 4, np.int64)
    iu = np.concatenate([iu, padi], 1)
    ju = np.concatenate([ju, padi], 1)
    icl = np.clip(iu, 0, _Hl - 1)
    jcl = np.clip(ju, 0, _Wl - 1)
    live = np.zeros((_Hl, _NRP), bool)
    live[:, :_NR] = True
    fl_iu = iu.reshape(-1)
    fl_ju = ju.reshape(-1)
    fl_ic = icl.reshape(-1)
    fl_jc = jcl.reshape(-1)
    fl_live = live.reshape(-1)

    ohit = np.zeros((_Hl, _NCELL), np.float32)
    ohjt = np.zeros((_Wl, _NCELL), np.float32)
    r = np.arange(_NCELL)[fl_live]
    ohit[fl_ic[fl_live], r] = 1.0
    ohjt[fl_jc[fl_live], r] = 1.0

    band = np.repeat(ts, _NRP)
    cxv = (fl_jc + 0.5) * _SCALE - 0.5 - 112.0
    cyl = (fl_ic + 0.5) * _SCALE - 0.5 - _SCALE * band
    sqc = (fl_iu - band).astype(np.float64) ** 2 + fl_ju.astype(
        np.float64) ** 2
    sqc = np.where(fl_live, sqc, 1e8)
    jm2 = np.where(fl_live, -2.0 * fl_ju, 0.0)

    wm = np.zeros((_NMAP, 16, _NCELL), np.float64)
    wm[0, 0] = -cxv * cxv
    wm[0, 1] = 2.0 * cxv
    wm[0, 4] = -1.0
    wm[1, 0] = -cxv * cyl
    wm[1, 1] = cyl
    wm[1, 2] = cxv
    wm[1, 5] = -1.0
    wm[2, 0] = -cyl * cyl
    wm[2, 2] = 2.0 * cyl
    wm[2, 6] = -1.0
    wm[3, 10] = -1.0
    wm[4, 0] = -1.0
    wm[5, 7] = 1.0
    wm[6, 8] = 1.0
    wm[7, 9] = 1.0
    wmts = wm.reshape(_NMAP * 16, _NCELL).astype(np.float32)

    rst = np.zeros((16, _NCELL), np.float64)
    rst[0] = sqc
    rst[3] = jm2
    rst[11] = 1.0
    rst[12] = -1.0
    rst = rst.astype(np.float32).reshape(16, _Hl, _NRP).transpose(1, 0, 2)

    flat = fl_ic * _Wl + fl_jc
    ohf = np.zeros((_Hl, _Hl * _Wl, _NRP), np.float32)
    cc = np.tile(np.arange(_NRP), _Hl)
    ohf[band[fl_live], flat[fl_live], cc[fl_live]] = 1.0

    p = np.arange(_P)
    xg = (p % _Wh).astype(np.float64)
    ylv = (p // _Wh).astype(np.float64)
    xv = xg - 112.0
    vcf = np.floor(xg / _SCALE)
    cfk = np.stack([np.ones(_P), xv, ylv, vcf, xv * xv, xv * ylv, ylv * ylv,
                    vcf * vcf, np.zeros(_P), np.zeros(_P),
                    np.zeros(_P)]).astype(np.float32)

    bh14 = _resize_mat(_Hl, _Hh)
    bwt = _resize_mat(_Wl, _Wh).T
    ah = _resize_mat(_Hh, _Hl)
    aht = ah.T.copy()
    ahb = np.zeros((_Hl, _Hl, _P), np.float32)
    for g in range(_Hl):
        ahb[g] = ah[16 * g + p // _Wh, :].T
    return ohit, ohjt, wmts, rst, ohf, cfk, bh14, bwt, aht, ahb


(_OHIT, _OHJT, _WMTS, _RST, _OHF, _CFK, _BH14, _BWT, _AHT,
 _AHB) = _build_static()


def _jbu_tile(guide3_ref, g672_ref, sxr_ref, syr_ref, thr_ref, srr_ref,
              feat_ref, ohf_ref, ohit_ref, ohjt_ref, wmts_ref, rst_ref,
              cfk_ref, bh_ref, bwt_ref, aht_ref, ahb_ref, out_ref,
              rl_scr, fb_scr, tt_scr):
    g = pl.program_id(0)
    f32 = jnp.float32
    hi = jax.lax.Precision.HIGHEST

    @pl.when(g == 0)
    def _build_tables():
        sxt = sxr_ref[...].T
        syt = syr_ref[...].T
        tht = thr_ref[...].T
        srt = srr_ref[...].T
        sxm = jnp.maximum(jnp.exp(sxt), 1e-6)
        sym = jnp.maximum(jnp.exp(syt), 1e-6)
        srm = jnp.maximum(jnp.exp(srt), 1e-6)
        isx = _LOG2E / (2.0 * sxm * sxm + 1e-8)
        isy = _LOG2E / (2.0 * sym * sym + 1e-8)
        isr = _LOG2E / (2.0 * srm * srm + 1e-8)
        th = jnp.pi * jnp.tanh(tht)
        ct = jnp.cos(th)
        st = jnp.sin(th)
        qa = ct * ct * isx + st * st * isy
        qb = 2.0 * ct * st * (isx - isy)
        qc = st * st * isx + ct * ct * isy

        bh = bh_ref[...]
        bwt = bwt_ref[...]
        glt = []
        for ch in range(3):
            gc = g672_ref[ch * _Hh:(ch + 1) * _Hh, :]
            glr = jnp.dot(bh, jnp.dot(gc, bwt, preferred_element_type=f32,
                                      precision=hi),
                          preferred_element_type=f32, precision=hi)
            glt.append(glr.T)
        glsq = glt[0] * glt[0] + glt[1] * glt[1] + glt[2] * glt[2]
        maps = [qa, qb, qc, isr, isr * glsq,
                2.0 * isr * glt[0], 2.0 * isr * glt[1], 2.0 * isr * glt[2]]

        ohit = ohit_ref[...]
        ohjt = ohjt_ref[...]
        ones14 = jnp.zeros((1, _Hl), f32) + 1.0
        rlt = jnp.zeros((16, _NCELL), f32)
        for k in range(_NMAP):
            t = jnp.dot(maps[k], ohit, preferred_element_type=f32,
                        precision=hi)
            gk = jnp.dot(ones14, t * ohjt, preferred_element_type=f32,
                         precision=hi)
            rlt = rlt + wmts_ref[16 * k:16 * (k + 1), :] * gk
        for t in range(_Hl):
            rl_scr[t, :, :] = rlt[:, _NRP * t:_NRP * (t + 1)]

        fa = jnp.concatenate(
            [feat_ref[...], jnp.zeros((1, _Hl * _Wl), f32) + 1.0,
             jnp.zeros((_NFA - 97, _Hl * _Wl), f32)], axis=0)
        for t in range(_Hl):
            fb_scr[t, :, :] = jnp.dot(fa, ohf_ref[t],
                                      preferred_element_type=f32,
                                      precision=hi)

        smax = jnp.exp(jnp.maximum(sxr_ref[...], syr_ref[...]))
        tsig = jnp.dot(smax, aht_ref[...], preferred_element_type=f32,
                       precision=hi)
        tt_scr[...] = jnp.concatenate([tsig] * _SCALE, axis=1)

    hp = jax.lax.Precision.HIGHEST
    f32 = jnp.float32
    ones14 = jnp.zeros((1, _Hl), f32) + 1.0
    sig = jnp.dot(ones14, ahb_ref[...].reshape(_Hl, _P) * tt_scr[...],
                  preferred_element_type=f32, precision=hi)
    rm = jnp.clip(jnp.ceil(_ALPHA_DYN * sig), 1, _R_MAX)
    r2 = rm * rm

    cfk = cfk_ref[...]
    gh = guide3_ref[...]
    gh0 = gh[0:1, :]
    gh1 = gh[1:2, :]
    gh2 = gh[2:3, :]
    ghsq = gh0 * gh0 + gh1 * gh1 + gh2 * gh2
    cf = jnp.concatenate([
        cfk[0:7, :], gh, ghsq, cfk[7:8, :], r2, cfk[8:11, :]], axis=0)

    rl = rl_scr[g]
    rs = rst_ref[...].reshape(16, _NRP)
    dn = (((0,), (0,)), ((), ()))
    lw2 = jax.lax.dot_general(rl, cf, dn, preferred_element_type=f32,
                              precision=hp)
    sqmr = jax.lax.dot_general(rs, cf, dn, preferred_element_type=f32,
                               precision=hp)
    lwm = lw2 - jnp.maximum(sqmr - 0.5, 0.0) * 1e30
    m = jnp.max(lwm, axis=0, keepdims=True)
    s = jnp.exp2(lwm - m)
    numa = jnp.dot(fb_scr[g], s, preferred_element_type=f32,
                   precision=hp)
    out_ref[...] = numa[0:96, :] / numa[96:97, :]


def kernel(feat_lr, guide_hr, sx_raw, sy_raw, th_raw, sr_raw):
    f32 = jnp.float32
    nc = feat_lr.shape[1]

    guide3 = guide_hr[0].astype(f32).reshape(3, _NPIX)
    g672 = guide_hr[0].astype(f32).reshape(3 * _Hh, _Wh)
    feat196 = feat_lr[0].astype(f32).reshape(nc, _Hl * _Wl)

    full = lambda g: (0, 0)
    out = pl.pallas_call(
        _jbu_tile,
        grid=(_Hl,),
        in_specs=[
            pl.BlockSpec((3, _P), lambda g: (0, g)),
            pl.BlockSpec((3 * _Hh, _Wh), full),
            pl.BlockSpec((_Hl, _Wl), full),
            pl.BlockSpec((_Hl, _Wl), full),
            pl.BlockSpec((_Hl, _Wl), full),
            pl.BlockSpec((_Hl, _Wl), full),
            pl.BlockSpec((nc, _Hl * _Wl), full),
            pl.BlockSpec((_Hl, _Hl * _Wl, _NRP), lambda g: (0, 0, 0)),
            pl.BlockSpec((_Hl, _NCELL), full),
            pl.BlockSpec((_Wl, _NCELL), full),
            pl.BlockSpec((_NMAP * 16, _NCELL), full),
            pl.BlockSpec((1, 16, _NRP), lambda g: (g, 0, 0)),
            pl.BlockSpec((11, _P), full),
            pl.BlockSpec((_Hl, _Hh), full),
            pl.BlockSpec((_Hh, _Wl), full),
            pl.BlockSpec((_Hl, _Hh), full),
            pl.BlockSpec((1, _Hl, _P), lambda g: (g, 0, 0)),
        ],
        out_specs=pl.BlockSpec((nc, _P), lambda g: (0, g)),
        out_shape=jax.ShapeDtypeStruct((nc, _NPIX), f32),
        scratch_shapes=[
            pltpu.VMEM((_Hl, 16, _NRP), f32),
            pltpu.VMEM((_Hl, _NFA, _NRP), f32),
            pltpu.VMEM((_Hl, _P), f32),
        ],
    )(guide3, g672, sx_raw[0, 0], sy_raw[0, 0], th_raw[0, 0], sr_raw[0, 0],
      feat196, jnp.asarray(_OHF), jnp.asarray(_OHIT), jnp.asarray(_OHJT),
      jnp.asarray(_WMTS), jnp.asarray(_RST), jnp.asarray(_CFK),
      jnp.asarray(_BH14), jnp.asarray(_BWT), jnp.asarray(_AHT),
      jnp.asarray(_AHB))

    return out.reshape(1, nc, _Hh, _Wh).astype(feat_lr.dtype)


# speedup vs baseline: 1.6122x; 1.4651x over previous
import numpy as np
import jax
import jax.numpy as jnp
from jax.experimental import pallas as pl
from jax.experimental.pallas import tpu as pltpu

_Hl, _Wl = 14, 14
_SCALE = 16
_R_MAX = 3
_ALPHA_DYN = 2.0
_Hh, _Wh = _Hl * _SCALE, _Wl * _SCALE
_NPIX = _Hh * _Wh
_EXT = _Wl + 2 * _R_MAX
_NR = 7 * _EXT
_NRP = 144
_P = _SCALE * _Wh
_NCELL = _Hl * _NRP
_NMAP = 8
_NFA = 104
_LOG2E = float(np.log2(np.e))


def _resize_mat(dst, src):
    m = np.zeros((dst, src), np.float32)
    for y in range(dst):
        u = (y + 0.5) * src / dst - 0.5
        i0 = int(np.floor(u))
        f = u - i0
        m[y, min(max(i0, 0), src - 1)] += 1.0 - f
        m[y, min(max(i0 + 1, 0), src - 1)] += f
    return m


def _build_static():
    dys = np.arange(-_R_MAX, _R_MAX + 1)
    ext_j = np.arange(-_R_MAX, _Wl + _R_MAX)
    ts = np.arange(_Hl)
    iu = np.broadcast_to(ts[:, None, None] + dys[None, :, None],
                         (_Hl, 7, _EXT)).reshape(_Hl, _NR)
    ju = np.broadcast_to(ext_j[None, None, :],
                         (_Hl, 7, _EXT)).reshape(_Hl, _NR)
    npad = _NRP - _NR
    padi = np.full((_Hl, npad), 10 ** 4, np.int64)
    iu = np.concatenate([iu, padi], 1)
    ju = np.concatenate([ju, padi], 1)
    icl = np.clip(iu, 0, _Hl - 1)
    jcl = np.clip(ju, 0, _Wl - 1)
    live = np.zeros((_Hl, _NRP), bool)
    live[:, :_NR] = True
    fl_iu = iu.reshape(-1)
    fl_ju = ju.reshape(-1)
    fl_ic = icl.reshape(-1)
    fl_jc = jcl.reshape(-1)
    fl_live = live.reshape(-1)

    ohit = np.zeros((_Hl, _NCELL), np.float32)
    ohjt = np.zeros((_Wl, _NCELL), np.float32)
    r = np.arange(_NCELL)[fl_live]
    ohit[fl_ic[fl_live], r] = 1.0
    ohjt[fl_jc[fl_live], r] = 1.0

    band = np.repeat(ts, _NRP)
    cxv = (fl_jc + 0.5) * _SCALE - 0.5 - 112.0
    cyl = (fl_ic + 0.5) * _SCALE - 0.5 - _SCALE * band
    sqc = (fl_iu - band).astype(np.float64) ** 2 + fl_ju.astype(
        np.float64) ** 2
    sqc = np.where(fl_live, sqc, 1e8)
    jm2 = np.where(fl_live, -2.0 * fl_ju, 0.0)

    wm = np.zeros((_NMAP, 16, _NCELL), np.float64)
    wm[0, 0] = -cxv * cxv
    wm[0, 1] = 2.0 * cxv
    wm[0, 4] = -1.0
    wm[1, 0] = -cxv * cyl
    wm[1, 1] = cyl
    wm[1, 2] = cxv
    wm[1, 5] = -1.0
    wm[2, 0] = -cyl * cyl
    wm[2, 2] = 2.0 * cyl
    wm[2, 6] = -1.0
    wm[3, 10] = -1.0
    wm[4, 0] = -1.0
    wm[5, 7] = 1.0
    wm[6, 8] = 1.0
    wm[7, 9] = 1.0
    wmts = wm.reshape(_NMAP * 16, _NCELL).astype(np.float32)

    dy2q = np.where(fl_live, (fl_iu - band).astype(np.float64) ** 2, 0.0)
    ju2q = np.where(fl_live, fl_ju.astype(np.float64) ** 2, 0.0)
    rst = np.zeros((16, _NCELL), np.float64)
    rst[0] = np.where(fl_live, dy2q / 4.0, float(2 ** 20))
    rst[3] = jm2 / 4.0
    rst[11] = 0.25
    rst[12] = -0.25
    rst[13] = ju2q / 4.0
    rst = rst.astype(np.float32).reshape(16, _Hl, _NRP).transpose(1, 0, 2)

    flat = fl_ic * _Wl + fl_jc
    ohf = np.zeros((_Hl, _Hl * _Wl, _NRP), np.float32)
    cc = np.tile(np.arange(_NRP), _Hl)
    ohf[band[fl_live], flat[fl_live], cc[fl_live]] = 1.0

    p = np.arange(_P)
    xg = (p % _Wh).astype(np.float64)
    ylv = (p // _Wh).astype(np.float64)
    xv = xg - 112.0
    vcf = np.floor(xg / _SCALE)
    cfk = np.stack([np.ones(_P), xv, ylv, vcf, xv * xv, xv * ylv, ylv * ylv,
                    vcf * vcf, np.ones(_P), np.zeros(_P),
                    np.zeros(_P)]).astype(np.float32)

    bh14 = _resize_mat(_Hl, _Hh)
    bwt = _resize_mat(_Wl, _Wh).T
    ah = _resize_mat(_Hh, _Hl)
    aht = ah.T.copy()
    ahb = np.zeros((_Hl, _Hl, _P), np.float32)
    for g in range(_Hl):
        ahb[g] = ah[16 * g + p // _Wh, :].T
    return ohit, ohjt, wmts, rst, ohf, cfk, bh14, bwt, aht, ahb


(_OHIT, _OHJT, _WMTS, _RST, _OHF, _CFK, _BH14, _BWT, _AHT,
 _AHB) = _build_static()


def _jbu_tile(guide3_ref, g672_ref, sxr_ref, syr_ref, thr_ref, srr_ref,
              feat_ref, ohf_ref, ohit_ref, ohjt_ref, wmts_ref, rst_ref,
              cfk_ref, bh_ref, bwt_ref, aht_ref, ahb_ref, out_ref,
              rl_scr, fb_scr, tt_scr):
    g = pl.program_id(0)
    f32 = jnp.float32
    hi = jax.lax.Precision.HIGHEST

    @pl.when(g == 0)
    def _build_tables():
        sxt = sxr_ref[...].T
        syt = syr_ref[...].T
        tht = thr_ref[...].T
        srt = srr_ref[...].T
        sxm = jnp.maximum(jnp.exp(sxt), 1e-6)
        sym = jnp.maximum(jnp.exp(syt), 1e-6)
        srm = jnp.maximum(jnp.exp(srt), 1e-6)
        isx = _LOG2E / (2.0 * sxm * sxm + 1e-8)
        isy = _LOG2E / (2.0 * sym * sym + 1e-8)
        isr = _LOG2E / (2.0 * srm * srm + 1e-8)
        th = jnp.pi * jnp.tanh(tht)
        ct = jnp.cos(th)
        st = jnp.sin(th)
        qa = ct * ct * isx + st * st * isy
        qb = 2.0 * ct * st * (isx - isy)
        qc = st * st * isx + ct * ct * isy

        bh = bh_ref[...]
        bwt = bwt_ref[...]
        glt = []
        for ch in range(3):
            gc = g672_ref[ch * _Hh:(ch + 1) * _Hh, :]
            glr = jnp.dot(bh, jnp.dot(gc, bwt, preferred_element_type=f32,
                                      precision=hi),
                          preferred_element_type=f32, precision=hi)
            glt.append(glr.T)
        glsq = glt[0] * glt[0] + glt[1] * glt[1] + glt[2] * glt[2]
        maps = [qa, qb, qc, isr, isr * glsq,
                2.0 * isr * glt[0], 2.0 * isr * glt[1], 2.0 * isr * glt[2]]

        ohit = ohit_ref[...]
        ohjt = ohjt_ref[...]
        ones14 = jnp.zeros((1, _Hl), f32) + 1.0
        rlt = jnp.zeros((16, _NCELL), f32)
        for k in range(_NMAP):
            t = jnp.dot(maps[k], ohit, preferred_element_type=f32,
                        precision=hi)
            gk = jnp.dot(ones14, t * ohjt, preferred_element_type=f32,
                         precision=hi)
            rlt = rlt + wmts_ref[16 * k:16 * (k + 1), :] * gk
        for t in range(_Hl):
            rl_scr[t, :, :] = rlt[:, _NRP * t:_NRP * (t + 1)]

        fa = jnp.concatenate(
            [feat_ref[...], jnp.zeros((1, _Hl * _Wl), f32) + 1.0,
             jnp.zeros((_NFA - 97, _Hl * _Wl), f32)], axis=0)
        for t in range(_Hl):
            fb_scr[t, :, :] = jnp.dot(fa, ohf_ref[t],
                                      preferred_element_type=f32,
                                      precision=hi).astype(jnp.bfloat16)

        smax = jnp.exp(jnp.maximum(sxr_ref[...], syr_ref[...]))
        tsig = jnp.dot(smax, aht_ref[...], preferred_element_type=f32,
                       precision=hi)
        tt_scr[...] = jnp.concatenate([tsig] * _SCALE, axis=1)

    f32 = jnp.float32
    ones14 = jnp.zeros((1, _Hl), f32) + 1.0
    sig = jnp.dot(ones14, ahb_ref[...].reshape(_Hl, _P) * tt_scr[...],
                  preferred_element_type=f32, precision=hi)
    rm = jnp.clip(jnp.ceil(_ALPHA_DYN * sig), 1, _R_MAX)
    r2 = rm * rm

    cfk = cfk_ref[...]
    gh = guide3_ref[...]
    gh0 = gh[0:1, :]
    gh1 = gh[1:2, :]
    gh2 = gh[2:3, :]
    ghsq = gh0 * gh0 + gh1 * gh1 + gh2 * gh2
    cf = jnp.concatenate([
        cfk[0:7, :], gh, ghsq, cfk[7:8, :], r2, cfk[8:11, :]], axis=0)

    bf16 = jnp.bfloat16
    dn = (((0,), (0,)), ((), ()))
    cfh = cf.astype(bf16)
    cfl = (cf - cfh.astype(f32)).astype(bf16)
    rl = rl_scr[g]
    rlh = rl.astype(bf16)
    rll = (rl - rlh.astype(f32)).astype(bf16)
    lw2 = (jax.lax.dot_general(rlh, cfh, dn, preferred_element_type=f32)
           + jax.lax.dot_general(rlh, cfl, dn, preferred_element_type=f32)
           + jax.lax.dot_general(rll, cfh, dn, preferred_element_type=f32))
    rs = rst_ref[...].reshape(16, _NRP).astype(bf16)
    sqmr = jax.lax.dot_general(rs, cfh, dn, preferred_element_type=f32)
    lwm = lw2 - jnp.maximum(sqmr - 0.125, 0.0) * 4e30
    m = jnp.max(lwm, axis=0, keepdims=True)
    s = jnp.exp2(lwm - m)
    sh = s.astype(bf16)
    sl = (s - sh.astype(f32)).astype(bf16)
    fb = fb_scr[g]
    numa = (jax.lax.dot_general(fb, sh, (((1,), (0,)), ((), ())),
                                preferred_element_type=f32)
            + jax.lax.dot_general(fb, sl, (((1,), (0,)), ((), ())),
                                  preferred_element_type=f32))
    out_ref[...] = numa[0:96, :] / numa[96:97, :]


def kernel(feat_lr, guide_hr, sx_raw, sy_raw, th_raw, sr_raw):
    f32 = jnp.float32
    nc = feat_lr.shape[1]

    guide3 = guide_hr[0].astype(f32).reshape(3, _NPIX)
    g672 = guide_hr[0].astype(f32).reshape(3 * _Hh, _Wh)
    feat196 = feat_lr[0].astype(f32).reshape(nc, _Hl * _Wl)

    full = lambda g: (0, 0)
    out = pl.pallas_call(
        _jbu_tile,
        grid=(_Hl,),
        in_specs=[
            pl.BlockSpec((3, _P), lambda g: (0, g)),
            pl.BlockSpec((3 * _Hh, _Wh), full),
            pl.BlockSpec((_Hl, _Wl), full),
            pl.BlockSpec((_Hl, _Wl), full),
            pl.BlockSpec((_Hl, _Wl), full),
            pl.BlockSpec((_Hl, _Wl), full),
            pl.BlockSpec((nc, _Hl * _Wl), full),
            pl.BlockSpec((_Hl, _Hl * _Wl, _NRP), lambda g: (0, 0, 0)),
            pl.BlockSpec((_Hl, _NCELL), full),
            pl.BlockSpec((_Wl, _NCELL), full),
            pl.BlockSpec((_NMAP * 16, _NCELL), full),
            pl.BlockSpec((1, 16, _NRP), lambda g: (g, 0, 0)),
            pl.BlockSpec((11, _P), full),
            pl.BlockSpec((_Hl, _Hh), full),
            pl.BlockSpec((_Hh, _Wl), full),
            pl.BlockSpec((_Hl, _Hh), full),
            pl.BlockSpec((1, _Hl, _P), lambda g: (g, 0, 0)),
        ],
        out_specs=pl.BlockSpec((nc, _P), lambda g: (0, g)),
        out_shape=jax.ShapeDtypeStruct((nc, _NPIX), f32),
        scratch_shapes=[
            pltpu.VMEM((_Hl, 16, _NRP), f32),
            pltpu.VMEM((_Hl, _NFA, _NRP), jnp.bfloat16),
            pltpu.VMEM((_Hl, _P), f32),
        ],
    )(guide3, g672, sx_raw[0, 0], sy_raw[0, 0], th_raw[0, 0], sr_raw[0, 0],
      feat196, jnp.asarray(_OHF), jnp.asarray(_OHIT), jnp.asarray(_OHJT),
      jnp.asarray(_WMTS), jnp.asarray(_RST), jnp.asarray(_CFK),
      jnp.asarray(_BH14), jnp.asarray(_BWT), jnp.asarray(_AHT),
      jnp.asarray(_AHB))

    return out.reshape(1, nc, _Hh, _Wh).astype(feat_lr.dtype)


# speedup vs baseline: 1.7591x; 1.0912x over previous
import numpy as np
import jax
import jax.numpy as jnp
from jax.experimental import pallas as pl
from jax.experimental.pallas import tpu as pltpu

_Hl, _Wl = 14, 14
_SCALE = 16
_R_MAX = 3
_ALPHA_DYN = 2.0
_Hh, _Wh = _Hl * _SCALE, _Wl * _SCALE
_NPIX = _Hh * _Wh
_EXT = _Wl + 2 * _R_MAX
_NR = 7 * _EXT
_NRP = 144
_P = _SCALE * _Wh
_NCELL = _Hl * _NRP
_NMAP = 8
_NFA = 104
_LOG2E = float(np.log2(np.e))


def _resize_mat(dst, src):
    m = np.zeros((dst, src), np.float32)
    for y in range(dst):
        u = (y + 0.5) * src / dst - 0.5
        i0 = int(np.floor(u))
        f = u - i0
        m[y, min(max(i0, 0), src - 1)] += 1.0 - f
        m[y, min(max(i0 + 1, 0), src - 1)] += f
    return m


def _build_static():
    dys = np.arange(-_R_MAX, _R_MAX + 1)
    ext_j = np.arange(-_R_MAX, _Wl + _R_MAX)
    ts = np.arange(_Hl)
    iu = np.broadcast_to(ts[:, None, None] + dys[None, :, None],
                         (_Hl, 7, _EXT)).reshape(_Hl, _NR)
    ju = np.broadcast_to(ext_j[None, None, :],
                         (_Hl, 7, _EXT)).reshape(_Hl, _NR)
    npad = _NRP - _NR
    padi = np.full((_Hl, npad), 10 ** 4, np.int64)
    iu = np.concatenate([iu, padi], 1)
    ju = np.concatenate([ju, padi], 1)
    icl = np.clip(iu, 0, _Hl - 1)
    jcl = np.clip(ju, 0, _Wl - 1)
    live = np.zeros((_Hl, _NRP), bool)
    live[:, :_NR] = True
    fl_iu = iu.reshape(-1)
    fl_ju = ju.reshape(-1)
    fl_ic = icl.reshape(-1)
    fl_jc = jcl.reshape(-1)
    fl_live = live.reshape(-1)

    ohit = np.zeros((_Hl, _NCELL), np.float32)
    ohjt = np.zeros((_Wl, _NCELL), np.float32)
    r = np.arange(_NCELL)[fl_live]
    ohit[fl_ic[fl_live], r] = 1.0
    ohjt[fl_jc[fl_live], r] = 1.0

    band = np.repeat(ts, _NRP)
    cxv = (fl_jc + 0.5) * _SCALE - 0.5 - 112.0
    cyl = (fl_ic + 0.5) * _SCALE - 0.5 - _SCALE * band
    sqc = (fl_iu - band).astype(np.float64) ** 2 + fl_ju.astype(
        np.float64) ** 2
    sqc = np.where(fl_live, sqc, 1e8)
    jm2 = np.where(fl_live, -2.0 * fl_ju, 0.0)

    wm = np.zeros((_NMAP, 16, _NCELL), np.float64)
    wm[0, 0] = -cxv * cxv
    wm[0, 1] = 2.0 * cxv
    wm[0, 4] = -1.0
    wm[1, 0] = -cxv * cyl
    wm[1, 1] = cyl
    wm[1, 2] = cxv
    wm[1, 5] = -1.0
    wm[2, 0] = -cyl * cyl
    wm[2, 2] = 2.0 * cyl
    wm[2, 6] = -1.0
    wm[3, 10] = -1.0
    wm[4, 0] = -1.0
    wm[5, 7] = 1.0
    wm[6, 8] = 1.0
    wm[7, 9] = 1.0
    wmts = wm.reshape(_NMAP * 16, _NCELL).astype(np.float32)

    dy2q = np.where(fl_live, (fl_iu - band).astype(np.float64) ** 2, 0.0)
    ju2q = np.where(fl_live, fl_ju.astype(np.float64) ** 2, 0.0)
    rst = np.zeros((16, _NCELL), np.float64)
    rst[0] = np.where(fl_live, dy2q / 4.0, float(2 ** 20))
    rst[3] = jm2 / 4.0
    rst[11] = 0.25
    rst[12] = -0.25
    rst[13] = ju2q / 4.0
    rst = rst.astype(np.float32).reshape(16, _Hl, _NRP).transpose(1, 0, 2)

    flat = fl_ic * _Wl + fl_jc
    ohf = np.zeros((_Hl, _Hl * _Wl, _NRP), np.float32)
    cc = np.tile(np.arange(_NRP), _Hl)
    ohf[band[fl_live], flat[fl_live], cc[fl_live]] = 1.0

    p = np.arange(_P)
    xg = (p % _Wh).astype(np.float64)
    ylv = (p // _Wh).astype(np.float64)
    xv = xg - 112.0
    vcf = np.floor(xg / _SCALE)
    cfk = np.stack([np.ones(_P), xv, ylv, vcf, xv * xv, xv * ylv, ylv * ylv,
                    vcf * vcf, np.ones(_P), np.zeros(_P),
                    np.zeros(_P)]).astype(np.float32)

    bh14 = _resize_mat(_Hl, _Hh)
    bwt = _resize_mat(_Wl, _Wh).T
    ah = _resize_mat(_Hh, _Hl)
    aht = ah.T.copy()
    ahb = np.zeros((_Hl, _Hl, _P), np.float32)
    for g in range(_Hl):
        ahb[g] = ah[16 * g + p // _Wh, :].T
    return ohit, ohjt, wmts, rst, ohf, cfk, bh14, bwt, aht, ahb


(_OHIT, _OHJT, _WMTS, _RST, _OHF, _CFK, _BH14, _BWT, _AHT,
 _AHB) = _build_static()


def _jbu_tile(guide3_ref, g672_ref, sxr_ref, syr_ref, thr_ref, srr_ref,
              feat_ref, ohf_ref, ohit_ref, ohjt_ref, wmts_ref, rst_ref,
              cfk_ref, bh_ref, bwt_ref, aht_ref, ahb_ref, out_ref,
              rl_scr, fb_scr, tt_scr):
    g = pl.program_id(0)
    f32 = jnp.float32
    hi = jax.lax.Precision.HIGHEST

    @pl.when(g == 0)
    def _build_tables():
        sxt = sxr_ref[...].T
        syt = syr_ref[...].T
        tht = thr_ref[...].T
        srt = srr_ref[...].T
        sxm = jnp.maximum(jnp.exp(sxt), 1e-6)
        sym = jnp.maximum(jnp.exp(syt), 1e-6)
        srm = jnp.maximum(jnp.exp(srt), 1e-6)
        isx = _LOG2E / (2.0 * sxm * sxm + 1e-8)
        isy = _LOG2E / (2.0 * sym * sym + 1e-8)
        isr = _LOG2E / (2.0 * srm * srm + 1e-8)
        th = jnp.pi * jnp.tanh(tht)
        ct = jnp.cos(th)
        st = jnp.sin(th)
        qa = ct * ct * isx + st * st * isy
        qb = 2.0 * ct * st * (isx - isy)
        qc = st * st * isx + ct * ct * isy

        bh = bh_ref[...]
        bwt = bwt_ref[...]
        glt = []
        for ch in range(3):
            gc = g672_ref[ch * _Hh:(ch + 1) * _Hh, :]
            glr = jnp.dot(bh, jnp.dot(gc, bwt, preferred_element_type=f32,
                                      precision=hi),
                          preferred_element_type=f32, precision=hi)
            glt.append(glr.T)
        glsq = glt[0] * glt[0] + glt[1] * glt[1] + glt[2] * glt[2]
        maps = [qa, qb, qc, isr, isr * glsq,
                2.0 * isr * glt[0], 2.0 * isr * glt[1], 2.0 * isr * glt[2]]

        ohit = ohit_ref[...]
        ohjt = ohjt_ref[...]
        ones14 = jnp.zeros((1, _Hl), f32) + 1.0
        rlt = jnp.zeros((16, _NCELL), f32)
        for k in range(_NMAP):
            t = jnp.dot(maps[k], ohit, preferred_element_type=f32,
                        precision=hi)
            gk = jnp.dot(ones14, t * ohjt, preferred_element_type=f32,
                         precision=hi)
            rlt = rlt + wmts_ref[16 * k:16 * (k + 1), :] * gk
        for t in range(_Hl):
            rl_scr[t, :, :] = rlt[:, _NRP * t:_NRP * (t + 1)]

        fa = jnp.concatenate(
            [feat_ref[...], jnp.zeros((1, _Hl * _Wl), f32) + 1.0,
             jnp.zeros((_NFA - 97, _Hl * _Wl), f32)],
            axis=0).astype(jnp.bfloat16)
        ohfb = ohf_ref[...].astype(jnp.bfloat16)
        for t in range(_Hl):
            fb_scr[t, :, :] = jnp.dot(fa, ohfb[t],
                                      preferred_element_type=f32
                                      ).astype(jnp.bfloat16)

        smax = jnp.exp(jnp.maximum(sxr_ref[...], syr_ref[...]))
        tsig = jnp.dot(smax, aht_ref[...], preferred_element_type=f32,
                       precision=hi)
        tt_scr[...] = jnp.concatenate([tsig] * _SCALE, axis=1)

    f32 = jnp.float32
    ones14 = jnp.zeros((1, _Hl), f32) + 1.0
    sig = jnp.dot(ones14, ahb_ref[...].reshape(_Hl, _P) * tt_scr[...],
                  preferred_element_type=f32, precision=hi)
    rm = jnp.clip(jnp.ceil(_ALPHA_DYN * sig), 1, _R_MAX)
    r2 = rm * rm

    cfk = cfk_ref[...]
    gh = guide3_ref[...]
    gh0 = gh[0:1, :]
    gh1 = gh[1:2, :]
    gh2 = gh[2:3, :]
    ghsq = gh0 * gh0 + gh1 * gh1 + gh2 * gh2
    cf = jnp.concatenate([
        cfk[0:7, :], gh, ghsq, cfk[7:8, :], r2, cfk[8:11, :]], axis=0)

    bf16 = jnp.bfloat16
    dn = (((0,), (0,)), ((), ()))
    cfh = cf.astype(bf16)
    cfl = (cf - cfh.astype(f32)).astype(bf16)
    rl = rl_scr[g]
    rlh = rl.astype(bf16)
    rll = (rl - rlh.astype(f32)).astype(bf16)
    lw2 = (jax.lax.dot_general(rlh, cfh, dn, preferred_element_type=f32)
           + jax.lax.dot_general(rlh, cfl, dn, preferred_element_type=f32)
           + jax.lax.dot_general(rll, cfh, dn, preferred_element_type=f32))
    rs = rst_ref[...].reshape(16, _NRP).astype(bf16)
    sqmr = jax.lax.dot_general(rs, cfh, dn, preferred_element_type=f32)
    lwm = lw2 - jnp.maximum(sqmr - 0.125, 0.0) * 4e30
    m = jnp.max(lwm, axis=0, keepdims=True)
    s = jnp.exp2(lwm - m)
    sh = s.astype(bf16)
    numa = jax.lax.dot_general(fb_scr[g], sh, (((1,), (0,)), ((), ())),
                               preferred_element_type=f32)
    out_ref[...] = numa[0:96, :] * (1.0 / numa[96:97, :])


def kernel(feat_lr, guide_hr, sx_raw, sy_raw, th_raw, sr_raw):
    f32 = jnp.float32
    nc = feat_lr.shape[1]

    guide3 = guide_hr[0].astype(f32).reshape(3, _NPIX)
    g672 = guide_hr[0].astype(f32).reshape(3 * _Hh, _Wh)
    feat196 = feat_lr[0].astype(f32).reshape(nc, _Hl * _Wl)

    full = lambda g: (0, 0)
    out = pl.pallas_call(
        _jbu_tile,
        grid=(_Hl,),
        in_specs=[
            pl.BlockSpec((3, _P), lambda g: (0, g)),
            pl.BlockSpec((3 * _Hh, _Wh), full),
            pl.BlockSpec((_Hl, _Wl), full),
            pl.BlockSpec((_Hl, _Wl), full),
            pl.BlockSpec((_Hl, _Wl), full),
            pl.BlockSpec((_Hl, _Wl), full),
            pl.BlockSpec((nc, _Hl * _Wl), full),
            pl.BlockSpec((_Hl, _Hl * _Wl, _NRP), lambda g: (0, 0, 0)),
            pl.BlockSpec((_Hl, _NCELL), full),
            pl.BlockSpec((_Wl, _NCELL), full),
            pl.BlockSpec((_NMAP * 16, _NCELL), full),
            pl.BlockSpec((1, 16, _NRP), lambda g: (g, 0, 0)),
            pl.BlockSpec((11, _P), full),
            pl.BlockSpec((_Hl, _Hh), full),
            pl.BlockSpec((_Hh, _Wl), full),
            pl.BlockSpec((_Hl, _Hh), full),
            pl.BlockSpec((1, _Hl, _P), lambda g: (g, 0, 0)),
        ],
        out_specs=pl.BlockSpec((nc, _P), lambda g: (0, g)),
        out_shape=jax.ShapeDtypeStruct((nc, _NPIX), f32),
        scratch_shapes=[
            pltpu.VMEM((_Hl, 16, _NRP), f32),
            pltpu.VMEM((_Hl, _NFA, _NRP), jnp.bfloat16),
            pltpu.VMEM((_Hl, _P), f32),
        ],
    )(guide3, g672, sx_raw[0, 0], sy_raw[0, 0], th_raw[0, 0], sr_raw[0, 0],
      feat196, jnp.asarray(_OHF), jnp.asarray(_OHIT), jnp.asarray(_OHJT),
      jnp.asarray(_WMTS), jnp.asarray(_RST), jnp.asarray(_CFK),
      jnp.asarray(_BH14), jnp.asarray(_BWT), jnp.asarray(_AHT),
      jnp.asarray(_AHB))

    return out.reshape(1, nc, _Hh, _Wh).astype(feat_lr.dtype)


# speedup vs baseline: 1.8816x; 1.0696x over previous
import numpy as np
import jax
import jax.numpy as jnp
from jax.experimental import pallas as pl
from jax.experimental.pallas import tpu as pltpu

_Hl, _Wl = 14, 14
_SCALE = 16
_R_MAX = 3
_ALPHA_DYN = 2.0
_Hh, _Wh = _Hl * _SCALE, _Wl * _SCALE
_NPIX = _Hh * _Wh
_EXT = _Wl + 2 * _R_MAX
_NR = 7 * _EXT
_NRP = 144
_P = _SCALE * _Wh
_NCELL = _Hl * _NRP
_NMAP = 8
_NFA = 104
_LOG2E = float(np.log2(np.e))


def _resize_mat(dst, src):
    m = np.zeros((dst, src), np.float32)
    for y in range(dst):
        u = (y + 0.5) * src / dst - 0.5
        i0 = int(np.floor(u))
        f = u - i0
        m[y, min(max(i0, 0), src - 1)] += 1.0 - f
        m[y, min(max(i0 + 1, 0), src - 1)] += f
    return m


def _build_static():
    dys = np.arange(-_R_MAX, _R_MAX + 1)
    ext_j = np.arange(-_R_MAX, _Wl + _R_MAX)
    ts = np.arange(_Hl)
    iu = np.broadcast_to(ts[:, None, None] + dys[None, :, None],
                         (_Hl, 7, _EXT)).reshape(_Hl, _NR)
    ju = np.broadcast_to(ext_j[None, None, :],
                         (_Hl, 7, _EXT)).reshape(_Hl, _NR)
    npad = _NRP - _NR
    padi = np.full((_Hl, npad), 10 ** 4, np.int64)
    iu = np.concatenate([iu, padi], 1)
    ju = np.concatenate([ju, padi], 1)
    icl = np.clip(iu, 0, _Hl - 1)
    jcl = np.clip(ju, 0, _Wl - 1)
    live = np.zeros((_Hl, _NRP), bool)
    live[:, :_NR] = True
    fl_iu = iu.reshape(-1)
    fl_ju = ju.reshape(-1)
    fl_ic = icl.reshape(-1)
    fl_jc = jcl.reshape(-1)
    fl_live = live.reshape(-1)

    ohit = np.zeros((_Hl, _NCELL), np.float32)
    ohjt = np.zeros((_Wl, _NCELL), np.float32)
    r = np.arange(_NCELL)[fl_live]
    ohit[fl_ic[fl_live], r] = 1.0
    ohjt[fl_jc[fl_live], r] = 1.0

    band = np.repeat(ts, _NRP)
    cxv = (fl_jc + 0.5) * _SCALE - 0.5 - 112.0
    cyl = (fl_ic + 0.5) * _SCALE - 0.5 - _SCALE * band
    sqc = (fl_iu - band).astype(np.float64) ** 2 + fl_ju.astype(
        np.float64) ** 2
    sqc = np.where(fl_live, sqc, 1e8)
    jm2 = np.where(fl_live, -2.0 * fl_ju, 0.0)

    wm = np.zeros((_NMAP, 16, _NCELL), np.float64)
    wm[0, 0] = -cxv * cxv
    wm[0, 1] = 2.0 * cxv
    wm[0, 4] = -1.0
    wm[1, 0] = -cxv * cyl
    wm[1, 1] = cyl
    wm[1, 2] = cxv
    wm[1, 5] = -1.0
    wm[2, 0] = -cyl * cyl
    wm[2, 2] = 2.0 * cyl
    wm[2, 6] = -1.0
    wm[3, 10] = -1.0
    wm[4, 0] = -1.0
    wm[5, 7] = 1.0
    wm[6, 8] = 1.0
    wm[7, 9] = 1.0
    wmts = wm.reshape(_NMAP * 16, _NCELL).astype(np.float32)

    dy2q = np.where(fl_live, (fl_iu - band).astype(np.float64) ** 2, 0.0)
    ju2q = np.where(fl_live, fl_ju.astype(np.float64) ** 2, 0.0)
    rst = np.zeros((16, _NCELL), np.float64)
    rst[0] = np.where(fl_live, dy2q / 4.0, float(2 ** 20))
    rst[3] = jm2 / 4.0
    rst[11] = 0.25
    rst[12] = -0.25
    rst[13] = ju2q / 4.0
    rst = rst.astype(np.float32).reshape(16, _Hl, _NRP).transpose(1, 0, 2)

    flat = fl_ic * _Wl + fl_jc
    ohf = np.zeros((_Hl, _Hl * _Wl, _NRP), np.float32)
    cc = np.tile(np.arange(_NRP), _Hl)
    ohf[band[fl_live], flat[fl_live], cc[fl_live]] = 1.0

    p = np.arange(_P)
    xg = (p % _Wh).astype(np.float64)
    ylv = (p // _Wh).astype(np.float64)
    xv = xg - 112.0
    vcf = np.floor(xg / _SCALE)
    cfk = np.stack([np.ones(_P), xv, ylv, vcf, xv * xv, xv * ylv, ylv * ylv,
                    vcf * vcf, np.ones(_P), np.zeros(_P),
                    np.zeros(_P)]).astype(np.float32)

    bh14 = _resize_mat(_Hl, _Hh)
    bwt = _resize_mat(_Wl, _Wh).T
    ah = _resize_mat(_Hh, _Hl)
    aht = ah.T.copy()
    ahb = np.zeros((_Hl, _Hl, _P), np.float32)
    for g in range(_Hl):
        ahb[g] = ah[16 * g + p // _Wh, :].T
    return ohit, ohjt, wmts, rst, ohf, cfk, bh14, bwt, aht, ahb


(_OHIT, _OHJT, _WMTS, _RST, _OHF, _CFK, _BH14, _BWT, _AHT,
 _AHB) = _build_static()


def _jbu_tile(guide3_ref, g672_ref, sxr_ref, syr_ref, thr_ref, srr_ref,
              feat_ref, ohf_ref, ohit_ref, ohjt_ref, wmts_ref, rst_ref,
              cfk_ref, bh_ref, bwt_ref, aht_ref, ahb_ref, out_ref,
              rl_scr, fb_scr, tt_scr):
    g = pl.program_id(0)
    f32 = jnp.float32
    hi = jax.lax.Precision.HIGHEST

    @pl.when(g == 0)
    def _build_tables():
        sxt = sxr_ref[...].T
        syt = syr_ref[...].T
        tht = thr_ref[...].T
        srt = srr_ref[...].T
        sxm = jnp.maximum(jnp.exp(sxt), 1e-6)
        sym = jnp.maximum(jnp.exp(syt), 1e-6)
        srm = jnp.maximum(jnp.exp(srt), 1e-6)
        isx = _LOG2E / (2.0 * sxm * sxm + 1e-8)
        isy = _LOG2E / (2.0 * sym * sym + 1e-8)
        isr = _LOG2E / (2.0 * srm * srm + 1e-8)
        th = jnp.pi * jnp.tanh(tht)
        ct = jnp.cos(th)
        st = jnp.sin(th)
        qa = ct * ct * isx + st * st * isy
        qb = 2.0 * ct * st * (isx - isy)
        qc = st * st * isx + ct * ct * isy

        bf = jnp.bfloat16

        def dot2(a, b):
            ah_ = a.astype(bf)
            al_ = (a - ah_.astype(f32)).astype(bf)
            return (jnp.dot(ah_, b, preferred_element_type=f32)
                    + jnp.dot(al_, b, preferred_element_type=f32))

        bh = bh_ref[...].astype(bf)
        bwt = bwt_ref[...].astype(bf)
        glt = []
        for ch in range(3):
            gc = g672_ref[ch * _Hh:(ch + 1) * _Hh, :]
            glr = dot2(dot2(gc, bwt).T, bh.T).T
            glt.append(glr.T)
        glsq = glt[0] * glt[0] + glt[1] * glt[1] + glt[2] * glt[2]
        maps = [qa, qb, qc, isr, isr * glsq,
                2.0 * isr * glt[0], 2.0 * isr * glt[1], 2.0 * isr * glt[2]]

        ohit = ohit_ref[...].astype(bf)
        ohjt = ohjt_ref[...]
        rlt = jnp.zeros((16, _NCELL), f32)
        for k in range(_NMAP):
            t = dot2(maps[k], ohit)
            gk = jnp.sum(t * ohjt, axis=0, keepdims=True)
            rlt = rlt + wmts_ref[16 * k:16 * (k + 1), :] * gk
        for t in range(_Hl):
            rl_scr[t, :, :] = rlt[:, _NRP * t:_NRP * (t + 1)]

        fa = jnp.concatenate(
            [feat_ref[...], jnp.zeros((1, _Hl * _Wl), f32) + 1.0,
             jnp.zeros((_NFA - 97, _Hl * _Wl), f32)],
            axis=0).astype(jnp.bfloat16)
        ohfb = ohf_ref[...].astype(jnp.bfloat16)
        for t in range(_Hl):
            fb_scr[t, :, :] = jnp.dot(fa, ohfb[t],
                                      preferred_element_type=f32
                                      ).astype(jnp.bfloat16)

        smax = jnp.exp(jnp.maximum(sxr_ref[...], syr_ref[...]))
        tsig = jnp.dot(smax, aht_ref[...], preferred_element_type=f32,
                       precision=hi)
        tt_scr[...] = jnp.concatenate([tsig] * _SCALE, axis=1)

    f32 = jnp.float32
    bf16 = jnp.bfloat16
    dn = (((0,), (0,)), ((), ()))
    ones14 = jnp.zeros((1, _Hl), f32) + 1.0
    cfk = cfk_ref[...]
    for h in range(2):
        band = 2 * g + h
        sl_ = slice(h * _P, (h + 1) * _P)
        sig = jnp.dot(ones14, ahb_ref[h] * tt_scr[...],
                      preferred_element_type=f32,
                      precision=jax.lax.Precision.HIGHEST)
        rm = jnp.clip(jnp.ceil(_ALPHA_DYN * sig), 1, _R_MAX)
        r2 = rm * rm

        gh = guide3_ref[:, sl_]
        gh0 = gh[0:1, :]
        gh1 = gh[1:2, :]
        gh2 = gh[2:3, :]
        ghsq = gh0 * gh0 + gh1 * gh1 + gh2 * gh2
        cf = jnp.concatenate([
            cfk[0:7, :], gh, ghsq, cfk[7:8, :], r2, cfk[8:11, :]], axis=0)
        cfh = cf.astype(bf16)
        cfl = (cf - cfh.astype(f32)).astype(bf16)
        rl = rl_scr[band]
        rlh = rl.astype(bf16)
        rll = (rl - rlh.astype(f32)).astype(bf16)
        lw2 = (jax.lax.dot_general(rlh, cfh, dn, preferred_element_type=f32)
               + jax.lax.dot_general(rlh, cfl, dn, preferred_element_type=f32)
               + jax.lax.dot_general(rll, cfh, dn,
                                     preferred_element_type=f32))
        rs = rst_ref[h].reshape(16, _NRP).astype(bf16)
        sqmr = jax.lax.dot_general(rs, cfh, dn, preferred_element_type=f32)
        lwm = lw2 - jnp.maximum(sqmr - 0.125, 0.0) * 4e30
        m = jnp.max(lwm, axis=0, keepdims=True)
        s = jnp.exp2(lwm - m)
        sh = s.astype(bf16)
        numa = jax.lax.dot_general(fb_scr[band], sh,
                                   (((1,), (0,)), ((), ())),
                                   preferred_element_type=f32)
        out_ref[:, sl_] = numa[0:96, :] * (1.0 / numa[96:97, :])


def kernel(feat_lr, guide_hr, sx_raw, sy_raw, th_raw, sr_raw):
    f32 = jnp.float32
    nc = feat_lr.shape[1]

    guide3 = guide_hr[0].astype(f32).reshape(3, _NPIX)
    g672 = guide_hr[0].astype(f32).reshape(3 * _Hh, _Wh)
    feat196 = feat_lr[0].astype(f32).reshape(nc, _Hl * _Wl)

    full = lambda g: (0, 0)
    out = pl.pallas_call(
        _jbu_tile,
        grid=(_Hl // 2,),
        in_specs=[
            pl.BlockSpec((3, 2 * _P), lambda g: (0, g)),
            pl.BlockSpec((3 * _Hh, _Wh), full),
            pl.BlockSpec((_Hl, _Wl), full),
            pl.BlockSpec((_Hl, _Wl), full),
            pl.BlockSpec((_Hl, _Wl), full),
            pl.BlockSpec((_Hl, _Wl), full),
            pl.BlockSpec((nc, _Hl * _Wl), full),
            pl.BlockSpec((_Hl, _Hl * _Wl, _NRP), lambda g: (0, 0, 0)),
            pl.BlockSpec((_Hl, _NCELL), full),
            pl.BlockSpec((_Wl, _NCELL), full),
            pl.BlockSpec((_NMAP * 16, _NCELL), full),
            pl.BlockSpec((2, 16, _NRP), lambda g: (g, 0, 0)),
            pl.BlockSpec((11, _P), full),
            pl.BlockSpec((_Hl, _Hh), full),
            pl.BlockSpec((_Hh, _Wl), full),
            pl.BlockSpec((_Hl, _Hh), full),
            pl.BlockSpec((2, _Hl, _P), lambda g: (g, 0, 0)),
        ],
        out_specs=pl.BlockSpec((nc, 2 * _P), lambda g: (0, g)),
        out_shape=jax.ShapeDtypeStruct((nc, _NPIX), f32),
        scratch_shapes=[
            pltpu.VMEM((_Hl, 16, _NRP), f32),
            pltpu.VMEM((_Hl, _NFA, _NRP), jnp.bfloat16),
            pltpu.VMEM((_Hl, _P), f32),
        ],
    )(guide3, g672, sx_raw[0, 0], sy_raw[0, 0], th_raw[0, 0], sr_raw[0, 0],
      feat196, jnp.asarray(_OHF), jnp.asarray(_OHIT), jnp.asarray(_OHJT),
      jnp.asarray(_WMTS), jnp.asarray(_RST), jnp.asarray(_CFK),
      jnp.asarray(_BH14), jnp.asarray(_BWT), jnp.asarray(_AHT),
      jnp.asarray(_AHB))

    return out.reshape(1, nc, _Hh, _Wh).astype(feat_lr.dtype)


# speedup vs baseline: 2.2524x; 1.1971x over previous
import numpy as np
import jax
import jax.numpy as jnp
from jax.experimental import pallas as pl
from jax.experimental.pallas import tpu as pltpu

_Hl, _Wl = 14, 14
_SCALE = 16
_R_MAX = 3
_ALPHA_DYN = 2.0
_Hh, _Wh = _Hl * _SCALE, _Wl * _SCALE
_NPIX = _Hh * _Wh
_EXT = _Wl + 2 * _R_MAX
_NR = 7 * _EXT
_NRP = 144
_P = _SCALE * _Wh
_NCELL = _Hl * _NRP
_NMAP = 8
_NFA = 104
_LOG2E = float(np.log2(np.e))


def _resize_mat(dst, src):
    m = np.zeros((dst, src), np.float32)
    for y in range(dst):
        u = (y + 0.5) * src / dst - 0.5
        i0 = int(np.floor(u))
        f = u - i0
        m[y, min(max(i0, 0), src - 1)] += 1.0 - f
        m[y, min(max(i0 + 1, 0), src - 1)] += f
    return m


def _build_static():
    dys = np.arange(-_R_MAX, _R_MAX + 1)
    ext_j = np.arange(-_R_MAX, _Wl + _R_MAX)
    ts = np.arange(_Hl)
    iu = np.broadcast_to(ts[:, None, None] + dys[None, :, None],
                         (_Hl, 7, _EXT)).reshape(_Hl, _NR)
    ju = np.broadcast_to(ext_j[None, None, :],
                         (_Hl, 7, _EXT)).reshape(_Hl, _NR)
    npad = _NRP - _NR
    padi = np.full((_Hl, npad), 10 ** 4, np.int64)
    iu = np.concatenate([iu, padi], 1)
    ju = np.concatenate([ju, padi], 1)
    icl = np.clip(iu, 0, _Hl - 1)
    jcl = np.clip(ju, 0, _Wl - 1)
    live = np.zeros((_Hl, _NRP), bool)
    live[:, :_NR] = True
    fl_iu = iu.reshape(-1)
    fl_ju = ju.reshape(-1)
    fl_ic = icl.reshape(-1)
    fl_jc = jcl.reshape(-1)
    fl_live = live.reshape(-1)

    ohit = np.zeros((_Hl, _NCELL), np.float32)
    ohjt = np.zeros((_Wl, _NCELL), np.float32)
    r = np.arange(_NCELL)[fl_live]
    ohit[fl_ic[fl_live], r] = 1.0
    ohjt[fl_jc[fl_live], r] = 1.0

    band = np.repeat(ts, _NRP)
    cxv = (fl_jc + 0.5) * _SCALE - 0.5 - 112.0
    cyl = (fl_ic + 0.5) * _SCALE - 0.5 - _SCALE * band
    sqc = (fl_iu - band).astype(np.float64) ** 2 + fl_ju.astype(
        np.float64) ** 2
    sqc = np.where(fl_live, sqc, 1e8)
    jm2 = np.where(fl_live, -2.0 * fl_ju, 0.0)

    wm = np.zeros((_NMAP, 16, _NCELL), np.float64)
    wm[0, 0] = -cxv * cxv
    wm[0, 1] = 2.0 * cxv
    wm[0, 4] = -1.0
    wm[1, 0] = -cxv * cyl
    wm[1, 1] = cyl
    wm[1, 2] = cxv
    wm[1, 5] = -1.0
    wm[2, 0] = -cyl * cyl
    wm[2, 2] = 2.0 * cyl
    wm[2, 6] = -1.0
    wm[3, 10] = -1.0
    wm[4, 0] = -1.0
    wm[5, 7] = 1.0
    wm[6, 8] = 1.0
    wm[7, 9] = 1.0
    wmts = wm.reshape(_NMAP * 16, _NCELL).astype(np.float32)

    dy2q = np.where(fl_live, (fl_iu - band).astype(np.float64) ** 2, 0.0)
    ju2q = np.where(fl_live, fl_ju.astype(np.float64) ** 2, 0.0)
    rst = np.zeros((16, _NCELL), np.float64)
    rst[0] = np.where(fl_live, dy2q / 4.0, float(2 ** 20))
    rst[3] = jm2 / 4.0
    rst[11] = 0.25
    rst[12] = -0.25
    rst[13] = ju2q / 4.0
    rst = rst.astype(np.float32).reshape(16, _Hl, _NRP).transpose(1, 0, 2)

    flat = fl_ic * _Wl + fl_jc
    ohf = np.zeros((_Hl, _Hl * _Wl, _NRP), np.float32)
    cc = np.tile(np.arange(_NRP), _Hl)
    ohf[band[fl_live], flat[fl_live], cc[fl_live]] = 1.0

    p = np.arange(_P)
    xg = (p % _Wh).astype(np.float64)
    ylv = (p // _Wh).astype(np.float64)
    xv = xg - 112.0
    vcf = np.floor(xg / _SCALE)
    cfk = np.stack([np.ones(_P), xv, ylv, vcf, xv * xv, xv * ylv, ylv * ylv,
                    vcf * vcf, np.ones(_P), np.zeros(_P),
                    np.zeros(_P)]).astype(np.float32)

    bh14 = _resize_mat(_Hl, _Hh)
    bwt = _resize_mat(_Wl, _Wh).T
    ah = _resize_mat(_Hh, _Hl)
    aht = ah.T.copy()
    ahb = np.zeros((_Hl, _Hl, _P), np.float32)
    for g in range(_Hl):
        ahb[g] = ah[16 * g + p // _Wh, :].T
    return ohit, ohjt, wmts, rst, ohf, cfk, bh14, bwt, aht, ahb


(_OHIT, _OHJT, _WMTS, _RST, _OHF, _CFK, _BH14, _BWT, _AHT,
 _AHB) = _build_static()


def _jbu_tile(guide3_ref, g672_ref, sxr_ref, syr_ref, thr_ref, srr_ref,
              feat_ref, ohf_ref, ohit_ref, ohjt_ref, wmts_ref, rst_ref,
              cfk_ref, bh_ref, bwt_ref, aht_ref, ahb_ref, out_ref,
              rl_scr, fb_scr, tt_scr):
    g = pl.program_id(0)
    f32 = jnp.float32
    hi = jax.lax.Precision.HIGHEST

    @pl.when(g == 0)
    def _build_tables():
        sxt = sxr_ref[...].T
        syt = syr_ref[...].T
        tht = thr_ref[...].T
        srt = srr_ref[...].T
        sxm = jnp.maximum(jnp.exp(sxt), 1e-6)
        sym = jnp.maximum(jnp.exp(syt), 1e-6)
        srm = jnp.maximum(jnp.exp(srt), 1e-6)
        isx = _LOG2E / (2.0 * sxm * sxm + 1e-8)
        isy = _LOG2E / (2.0 * sym * sym + 1e-8)
        isr = _LOG2E / (2.0 * srm * srm + 1e-8)
        th = jnp.pi * jnp.tanh(tht)
        ct = jnp.cos(th)
        st = jnp.sin(th)
        qa = ct * ct * isx + st * st * isy
        qb = 2.0 * ct * st * (isx - isy)
        qc = st * st * isx + ct * ct * isy

        bf = jnp.bfloat16

        def dot2(a, b):
            ah_ = a.astype(bf)
            al_ = (a - ah_.astype(f32)).astype(bf)
            return (jnp.dot(ah_, b, preferred_element_type=f32)
                    + jnp.dot(al_, b, preferred_element_type=f32))

        bh = bh_ref[...].astype(bf)
        bwt = bwt_ref[...].astype(bf)
        glt = []
        for ch in range(3):
            gc = g672_ref[ch * _Hh:(ch + 1) * _Hh, :]
            glr = dot2(dot2(gc, bwt).T, bh.T).T
            glt.append(glr.T)
        glsq = glt[0] * glt[0] + glt[1] * glt[1] + glt[2] * glt[2]
        maps = [qa, qb, qc, isr, isr * glsq,
                2.0 * isr * glt[0], 2.0 * isr * glt[1], 2.0 * isr * glt[2]]

        ohit = ohit_ref[...].astype(bf)
        ohjt = ohjt_ref[...]
        rlt = jnp.zeros((16, _NCELL), f32)
        for k in range(_NMAP):
            t = dot2(maps[k], ohit)
            gk = jnp.sum(t * ohjt, axis=0, keepdims=True)
            rlt = rlt + wmts_ref[16 * k:16 * (k + 1), :] * gk
        for t in range(_Hl):
            rl_scr[t, :, :] = rlt[:, _NRP * t:_NRP * (t + 1)]

        fa = jnp.concatenate(
            [feat_ref[...], jnp.zeros((1, _Hl * _Wl), f32) + 1.0,
             jnp.zeros((_NFA - 97, _Hl * _Wl), f32)],
            axis=0).astype(jnp.bfloat16)
        ohfb = ohf_ref[...].astype(jnp.bfloat16)
        for t in range(_Hl):
            fb_scr[t, :, :] = jnp.dot(fa, ohfb[t],
                                      preferred_element_type=f32
                                      ).astype(jnp.bfloat16)

        smax = jnp.exp(jnp.maximum(sxr_ref[...], syr_ref[...]))
        tsig = jnp.dot(smax, aht_ref[...], preferred_element_type=f32,
                       precision=hi)
        tt_scr[...] = jnp.concatenate([tsig] * _SCALE, axis=1)

    f32 = jnp.float32
    bf16 = jnp.bfloat16
    dn = (((0,), (0,)), ((), ()))
    ones14 = jnp.zeros((1, _Hl), f32) + 1.0
    cfk = cfk_ref[...]
    for h in range(2):
        band = 2 * g + h
        sl_ = slice(h * _P, (h + 1) * _P)
        sig = jnp.dot(ones14, ahb_ref[h] * tt_scr[...],
                      preferred_element_type=f32,
                      precision=jax.lax.Precision.HIGHEST)
        rm = jnp.clip(jnp.ceil(_ALPHA_DYN * sig), 1, _R_MAX)
        r2 = rm * rm

        gh = guide3_ref[:, sl_]
        gh0 = gh[0:1, :]
        gh1 = gh[1:2, :]
        gh2 = gh[2:3, :]
        ghsq = gh0 * gh0 + gh1 * gh1 + gh2 * gh2
        cf = jnp.concatenate([
            cfk[0:7, :], gh, ghsq, cfk[7:8, :], r2, cfk[8:11, :]], axis=0)
        cfh = cf.astype(bf16)
        cfl = (cf - cfh.astype(f32)).astype(bf16)
        rl = rl_scr[band]
        rlh = rl.astype(bf16)
        rll = (rl - rlh.astype(f32)).astype(bf16)
        zb = jnp.zeros((16, _NRP), bf16)
        rs = rst_ref[h].reshape(16, _NRP).astype(bf16)
        lhs = jnp.concatenate(
            [jnp.concatenate([rlh, rlh, rll], axis=0),
             jnp.concatenate([rs, zb, zb], axis=0)], axis=1)
        rhs = jnp.concatenate([cfh, cfl, cfh], axis=0)
        both = jax.lax.dot_general(lhs, rhs, dn, preferred_element_type=f32)
        lw2 = both[0:_NRP, :]
        sqmr = both[_NRP:2 * _NRP, :]
        lwm = lw2 - jnp.maximum(sqmr - 0.125, 0.0) * 4e30
        m = jnp.max(lwm, axis=0, keepdims=True)
        s = jnp.exp2(lwm - m)
        sh = s.astype(bf16)
        numa = jax.lax.dot_general(fb_scr[band], sh,
                                   (((1,), (0,)), ((), ())),
                                   preferred_element_type=f32)
        out_ref[:, sl_] = numa[0:96, :] * (1.0 / numa[96:97, :])


def kernel(feat_lr, guide_hr, sx_raw, sy_raw, th_raw, sr_raw):
    f32 = jnp.float32
    nc = feat_lr.shape[1]

    guide3 = guide_hr[0].astype(f32).reshape(3, _NPIX)
    g672 = guide_hr[0].astype(f32).reshape(3 * _Hh, _Wh)
    feat196 = feat_lr[0].astype(f32).reshape(nc, _Hl * _Wl)

    full = lambda g: (0, 0)
    out = pl.pallas_call(
        _jbu_tile,
        grid=(_Hl // 2,),
        in_specs=[
            pl.BlockSpec((3, 2 * _P), lambda g: (0, g)),
            pl.BlockSpec((3 * _Hh, _Wh), full),
            pl.BlockSpec((_Hl, _Wl), full),
            pl.BlockSpec((_Hl, _Wl), full),
            pl.BlockSpec((_Hl, _Wl), full),
            pl.BlockSpec((_Hl, _Wl), full),
            pl.BlockSpec((nc, _Hl * _Wl), full),
            pl.BlockSpec((_Hl, _Hl * _Wl, _NRP), lambda g: (0, 0, 0)),
            pl.BlockSpec((_Hl, _NCELL), full),
            pl.BlockSpec((_Wl, _NCELL), full),
            pl.BlockSpec((_NMAP * 16, _NCELL), full),
            pl.BlockSpec((2, 16, _NRP), lambda g: (g, 0, 0)),
            pl.BlockSpec((11, _P), full),
            pl.BlockSpec((_Hl, _Hh), full),
            pl.BlockSpec((_Hh, _Wl), full),
            pl.BlockSpec((_Hl, _Hh), full),
            pl.BlockSpec((2, _Hl, _P), lambda g: (g, 0, 0)),
        ],
        out_specs=pl.BlockSpec((nc, 2 * _P), lambda g: (0, g)),
        out_shape=jax.ShapeDtypeStruct((nc, _NPIX), f32),
        scratch_shapes=[
            pltpu.VMEM((_Hl, 16, _NRP), f32),
            pltpu.VMEM((_Hl, _NFA, _NRP), jnp.bfloat16),
            pltpu.VMEM((_Hl, _P), f32),
        ],
    )(guide3, g672, sx_raw[0, 0], sy_raw[0, 0], th_raw[0, 0], sr_raw[0, 0],
      feat196, jnp.asarray(_OHF), jnp.asarray(_OHIT), jnp.asarray(_OHJT),
      jnp.asarray(_WMTS), jnp.asarray(_RST), jnp.asarray(_CFK),
      jnp.asarray(_BH14), jnp.asarray(_BWT), jnp.asarray(_AHT),
      jnp.asarray(_AHB))

    return out.reshape(1, nc, _Hh, _Wh).astype(feat_lr.dtype)
